# Initial kernel scaffold; baseline (speedup 1.0000x reference)
import jax
import jax.numpy as jnp
from jax.experimental import pallas as pl


def kernel(x, edge_index, W1, b1, W2, b2):
    raise NotImplementedError("write your pallas kernel here")


# speedup vs baseline: 12.1537x; 12.1537x over previous
import functools

import jax
import jax.numpy as jnp
from jax import lax
from jax.experimental import pallas as pl
from jax.experimental.pallas import tpu as pltpu
from jax.experimental.pallas import tpu_sc as plsc

N_NODES = 100000
N_EDGES = 1600000
F_IN = 128
HID = 32
NCLS = 40

LANES = 128
ROWS = 12544
ROWS_PER_TILE = 784
CHUNK = 8
HALF = N_NODES // 2
TR = 51200
TPT = TR // 16
SEGS = TPT // 128
DUMMY = HALF

_MESH = dict(core_axis_name="c", subcore_axis_name="s")


def _make_deg_kernel():
    mesh = plsc.VectorSubcoreMesh(**_MESH)

    @functools.partial(
        pl.kernel,
        mesh=mesh,
        compiler_params=pltpu.CompilerParams(use_tc_tiling_on_sc=False),
        out_type=jax.ShapeDtypeStruct((2 * TR,), jnp.float32),
        scratch_types=[
            pltpu.VMEM((CHUNK, LANES), jnp.int32),
            pltpu.VMEM((CHUNK, LANES), jnp.int32),
            pltpu.VMEM((LANES,), jnp.float32),
            pltpu.VMEM((LANES,), jnp.float32),
            pltpu.VMEM_SHARED((TR,), jnp.float32),
        ],
    )
    def deg_kernel(dst_hbm, out_hbm, dst_v, loc_v, ones_v, zb_v, table):
        c = lax.axis_index("c")
        s = lax.axis_index("s")
        base = c * HALF
        for q in range(LANES // 16):
            zb_v[pl.ds(q * 16, 16)] = jnp.zeros((16,), jnp.float32)
            ones_v[pl.ds(q * 16, 16)] = jnp.ones((16,), jnp.float32)
        tb = s * TPT
        for k in range(SEGS):
            pltpu.sync_copy(zb_v, table.at[pl.ds(tb + k * 128, 128)])
        plsc.subcore_barrier()

        row0 = s * ROWS_PER_TILE

        def body(it, carry):
            r = row0 + it * CHUNK
            pltpu.sync_copy(dst_hbm.at[pl.ds(r, CHUNK)], dst_v)
            for i in range(CHUNK):
                for q in range(LANES // 16):
                    d = dst_v[i, pl.ds(q * 16, 16)]
                    l = d - base
                    ok = (l >= 0) & (l < HALF)
                    loc_v[i, pl.ds(q * 16, 16)] = jnp.where(ok, l, DUMMY)
            for i in range(CHUNK):
                pltpu.sync_copy(ones_v, table.at[loc_v.at[i]], add=True)
            return carry

        lax.fori_loop(0, ROWS_PER_TILE // CHUNK, body, 0)
        plsc.subcore_barrier()
        for k in range(SEGS):
            pltpu.sync_copy(table.at[pl.ds(tb + k * 128, 128)],
                            out_hbm.at[pl.ds(c * TR + tb + k * 128, 128)])

    return deg_kernel


def _make_agg_kernel(D):
    mesh = plsc.VectorSubcoreMesh(**_MESH)

    @functools.partial(
        pl.kernel,
        mesh=mesh,
        compiler_params=pltpu.CompilerParams(use_tc_tiling_on_sc=False),
        out_type=jax.ShapeDtypeStruct((2, TR, D), jnp.float32),
        scratch_types=[
            pltpu.VMEM((CHUNK, LANES), jnp.int32),
            pltpu.VMEM((CHUNK, LANES), jnp.int32),
            pltpu.VMEM((CHUNK, LANES), jnp.int32),
            pltpu.VMEM((LANES, D), jnp.float32),
            pltpu.VMEM((LANES, D), jnp.float32),
            pltpu.VMEM_SHARED((TR, D), jnp.float32),
            pltpu.SemaphoreType.DMA,
            pltpu.SemaphoreType.DMA,
        ],
    )
    def agg_kernel(hs_hbm, src_hbm, dst_hbm, zeros_hbm, out_hbm,
                   src_v, dst_v, loc_v, buf0, buf1, table, sem0, sem1):
        c = lax.axis_index("c")
        s = lax.axis_index("s")
        base = c * HALF
        tb = s * TPT
        for k in range(SEGS):
            pltpu.sync_copy(zeros_hbm, table.at[pl.ds(tb + k * 128, 128)])
        plsc.subcore_barrier()

        row0 = s * ROWS_PER_TILE
        bufs = (buf0, buf1)
        sems = (sem0, sem1)

        def body(it, carry):
            r = row0 + it * CHUNK
            pltpu.sync_copy(src_hbm.at[pl.ds(r, CHUNK)], src_v)
            pltpu.sync_copy(dst_hbm.at[pl.ds(r, CHUNK)], dst_v)
            for i in range(CHUNK):
                for q in range(LANES // 16):
                    d = dst_v[i, pl.ds(q * 16, 16)]
                    l = d - base
                    ok = (l >= 0) & (l < HALF)
                    loc_v[i, pl.ds(q * 16, 16)] = jnp.where(ok, l, DUMMY)
            cps = [None] * CHUNK
            cps[0] = pltpu.async_copy(hs_hbm.at[src_v.at[0]], bufs[0], sems[0])
            for i in range(CHUNK):
                if i + 1 < CHUNK:
                    cps[i + 1] = pltpu.async_copy(
                        hs_hbm.at[src_v.at[i + 1]],
                        bufs[(i + 1) % 2], sems[(i + 1) % 2])
                cps[i].wait()
                pltpu.sync_copy(bufs[i % 2], table.at[loc_v.at[i]], add=True)
            return carry

        lax.fori_loop(0, ROWS_PER_TILE // CHUNK, body, 0)
        plsc.subcore_barrier()
        for k in range(SEGS):
            pltpu.sync_copy(table.at[pl.ds(tb + k * 128, 128)],
                            out_hbm.at[c, pl.ds(tb + k * 128, 128)])

    return agg_kernel


_deg_call = _make_deg_kernel()
_agg32_call = _make_agg_kernel(HID)


BLK = 2000


def _pre_body(x_ref, w_ref, deg_ref, hs_ref, s_ref):
    s = lax.rsqrt(deg_ref[0] + 1.0)
    h = jnp.dot(x_ref[...], w_ref[...], preferred_element_type=jnp.float32)
    hs_ref[...] = h * s
    s_ref[...] = s


def _pre_call(x, W1, deg3):
    return pl.pallas_call(
        _pre_body,
        grid=(2, HALF // BLK),
        in_specs=[
            pl.BlockSpec((BLK, F_IN), lambda i, j: (i * (HALF // BLK) + j, 0)),
            pl.BlockSpec((F_IN, HID), lambda i, j: (0, 0)),
            pl.BlockSpec((1, BLK, 1), lambda i, j: (i, j, 0)),
        ],
        out_specs=[
            pl.BlockSpec((BLK, HID), lambda i, j: (i * (HALF // BLK) + j, 0)),
            pl.BlockSpec((BLK, 1), lambda i, j: (i * (HALF // BLK) + j, 0)),
        ],
        out_shape=[
            jax.ShapeDtypeStruct((N_NODES, HID), jnp.float32),
            jax.ShapeDtypeStruct((N_NODES, 1), jnp.float32),
        ],
    )(x, W1, deg3)


def _mid_body(agg_ref, hs_ref, s_ref, b_ref, out_ref):
    s = s_ref[...]
    z = jnp.maximum(s * (agg_ref[0] + hs_ref[...]) + b_ref[...], 0.0)
    out_ref[...] = z * s


def _mid_call(agg1, hs1, sinv, b1r):
    return pl.pallas_call(
        _mid_body,
        grid=(2, HALF // BLK),
        in_specs=[
            pl.BlockSpec((1, BLK, HID), lambda i, j: (i, j, 0)),
            pl.BlockSpec((BLK, HID), lambda i, j: (i * (HALF // BLK) + j, 0)),
            pl.BlockSpec((BLK, 1), lambda i, j: (i * (HALF // BLK) + j, 0)),
            pl.BlockSpec((1, HID), lambda i, j: (0, 0)),
        ],
        out_specs=pl.BlockSpec((BLK, HID),
                               lambda i, j: (i * (HALF // BLK) + j, 0)),
        out_shape=jax.ShapeDtypeStruct((N_NODES, HID), jnp.float32),
    )(agg1, hs1, sinv, b1r)


def _post_body(agg_ref, zs_ref, s_ref, w_ref, b_ref, out_ref):
    s = s_ref[...]
    u = s * (agg_ref[0] + zs_ref[...])
    h2 = jnp.dot(u, w_ref[...], preferred_element_type=jnp.float32)
    z = jnp.maximum(h2 + b_ref[...], 0.0)
    m = jnp.max(z, axis=1, keepdims=True)
    lse = jnp.log(jnp.sum(jnp.exp(z - m), axis=1, keepdims=True)) + m
    out_ref[...] = z - lse


def _post_call(agg2, zs, sinv, W2, b2r):
    return pl.pallas_call(
        _post_body,
        grid=(2, HALF // BLK),
        in_specs=[
            pl.BlockSpec((1, BLK, HID), lambda i, j: (i, j, 0)),
            pl.BlockSpec((BLK, HID), lambda i, j: (i * (HALF // BLK) + j, 0)),
            pl.BlockSpec((BLK, 1), lambda i, j: (i * (HALF // BLK) + j, 0)),
            pl.BlockSpec((HID, NCLS), lambda i, j: (0, 0)),
            pl.BlockSpec((1, NCLS), lambda i, j: (0, 0)),
        ],
        out_specs=pl.BlockSpec((BLK, NCLS),
                               lambda i, j: (i * (HALF // BLK) + j, 0)),
        out_shape=jax.ShapeDtypeStruct((N_NODES, NCLS), jnp.float32),
    )(agg2, zs, sinv, W2, b2r)


def kernel(x, edge_index, W1, b1, W2, b2):
    pad = ROWS * LANES - N_EDGES
    srcp = jnp.concatenate(
        [edge_index[0], jnp.zeros((pad,), jnp.int32)]).reshape(ROWS, LANES)
    dstp = jnp.concatenate(
        [edge_index[1], jnp.full((pad,), N_NODES, jnp.int32)]).reshape(ROWS, LANES)

    deg2 = _deg_call(dstp).reshape(2, TR)
    deg3 = deg2[:, :HALF].reshape(2, HALF, 1)

    hs1, sinv = _pre_call(x, W1, deg3)
    zeros32 = jnp.zeros((LANES, HID), jnp.float32)
    agg1 = _agg32_call(hs1, srcp, dstp, zeros32)

    zs = _mid_call(agg1, hs1, sinv, b1.reshape(1, HID))
    agg2 = _agg32_call(zs, srcp, dstp, zeros32)

    return _post_call(agg2, zs, sinv, W2, b2.reshape(1, NCLS))
